# Initial kernel scaffold; baseline (speedup 1.0000x reference)
import jax
import jax.numpy as jnp
from jax.experimental import pallas as pl


def kernel(data, numpoints, W0, b0, W_blocks, b_blocks, Wf, bf):
    raise NotImplementedError("write your pallas kernel here")


# speedup vs baseline: 10.1311x; 10.1311x over previous
import functools

import jax
import jax.numpy as jnp
from jax import lax
from jax.experimental import pallas as pl
from jax.experimental.pallas import tpu as pltpu
from jax.experimental.pallas import tpu_sc as plsc

K = 16
CH = 64
RB = 256
PPT = 128
CPT = 16
BIG = 3.0e38


def _knn_body(x_blk_ref, x_full_ref, sqc_ref, sqr_ref, idx_ref):
    b = pl.program_id(0)
    n = x_full_ref.shape[1]
    x_blk = x_blk_ref[0]
    x_full = x_full_ref[0]

    g = lax.dot_general(x_blk, x_full, (((1,), (1,)), ((), ())),
                        preferred_element_type=jnp.float32)
    d = (sqc_ref[0] - 2.0 * g) + sqr_ref[0]

    iota = lax.broadcasted_iota(jnp.int32, (RB, n), 1)
    kiota = lax.broadcasted_iota(jnp.int32, (RB, K), 1)
    idx_blk = jnp.zeros((RB, K), jnp.int32)
    for k in range(K):
        dmin = jnp.min(d, axis=1, keepdims=True)
        cand = jnp.where(d == dmin, iota, n)
        imin = jnp.min(cand, axis=1, keepdims=True)
        idx_blk = jnp.where(kiota == k, imin, idx_blk)
        d = jnp.where(iota == imin, BIG, d)
    idx_ref[0, 0] = idx_blk + b * n


def _knn(x, sq_col, sq_row):
    b, n, _ = x.shape
    nrb = n // RB
    return pl.pallas_call(
        _knn_body,
        grid=(b, nrb),
        in_specs=[
            pl.BlockSpec((1, RB, CH), lambda i, r: (i, r, 0)),
            pl.BlockSpec((1, n, CH), lambda i, r: (i, 0, 0)),
            pl.BlockSpec((1, RB, 1), lambda i, r: (i, r, 0)),
            pl.BlockSpec((1, 1, n), lambda i, r: (i, 0, 0)),
        ],
        out_specs=pl.BlockSpec((1, 1, RB, K), lambda i, r: (i, r, 0, 0)),
        out_shape=jax.ShapeDtypeStruct((b, nrb, RB, K), jnp.int32),
    )(x, x, sq_col, sq_row)


def _sc_body(x_hbm, idx_hbm, out_hbm, idxc_v, rows_v, sem):
    wid = lax.axis_index("s") * 2 + lax.axis_index("c")
    base = wid * PPT

    def chunk(c, carry):
        pltpu.sync_copy(idx_hbm.at[wid, c], idxc_v)
        pltpu.async_copy(x_hbm.at[idxc_v], rows_v, sem).wait()
        pltpu.sync_copy(rows_v, out_hbm.at[pl.ds((base + c * 8) * K, PPT)])
        return carry

    lax.fori_loop(0, CPT, chunk, 0)


@functools.lru_cache(maxsize=None)
def _sc_gather(bn):
    return pl.kernel(
        _sc_body,
        out_type=jax.ShapeDtypeStruct((bn * K, CH), jnp.float32),
        compiler_params=pltpu.CompilerParams(use_tc_tiling_on_sc=False),
        mesh=plsc.VectorSubcoreMesh(core_axis_name="c", subcore_axis_name="s"),
        scratch_types=[
            pltpu.VMEM((PPT,), jnp.int32),
            pltpu.VMEM((PPT, CH), jnp.float32),
            pltpu.SemaphoreType.DMA,
        ],
    )


def _gather_rows(x2, idx3):
    return _sc_gather(x2.shape[0])(x2, idx3)


def _ec_body(x_ref, nbr_ref, w_ref, b_ref, h_ref):
    xc = x_ref[...]
    m = None
    for j in range(K):
        nbj = nbr_ref[:, j, :]
        feat = jnp.concatenate([xc, nbj - xc], axis=1)
        hj = lax.dot_general(feat, w_ref[...], (((1,), (0,)), ((), ())),
                             preferred_element_type=jnp.float32) + b_ref[...]
        m = hj if m is None else jnp.maximum(m, hj)
    h_ref[...] = jnp.maximum(m, 0.0)


def _edge_conv(x2, nbr3, w, b2):
    bn = x2.shape[0]
    return pl.pallas_call(
        _ec_body,
        grid=(bn // RB,),
        in_specs=[
            pl.BlockSpec((RB, CH), lambda r: (r, 0)),
            pl.BlockSpec((RB, K, CH), lambda r: (r, 0, 0)),
            pl.BlockSpec((2 * CH, CH), lambda r: (0, 0)),
            pl.BlockSpec((1, CH), lambda r: (0, 0)),
        ],
        out_specs=pl.BlockSpec((RB, CH), lambda r: (r, 0)),
        out_shape=jax.ShapeDtypeStruct((bn, CH), jnp.float32),
    )(x2, nbr3, w, b2)


def _final_body(cat_ref, wf_ref, bf_ref, out_ref):
    r = pl.program_id(1)
    g = lax.dot_general(cat_ref[0], wf_ref[...], (((1,), (0,)), ((), ())),
                        preferred_element_type=jnp.float32)
    g = jnp.maximum(g + bf_ref[...], 0.0)
    m = jnp.max(g, axis=0, keepdims=True)

    @pl.when(r == 0)
    def _():
        out_ref[0] = m

    @pl.when(r > 0)
    def _():
        out_ref[0] = jnp.maximum(out_ref[0], m)


def _final(cat, wf, bf2):
    b, n, cin = cat.shape
    emb = wf.shape[1]
    return pl.pallas_call(
        _final_body,
        grid=(b, n // RB),
        in_specs=[
            pl.BlockSpec((1, RB, cin), lambda i, r: (i, r, 0)),
            pl.BlockSpec((cin, emb), lambda i, r: (0, 0)),
            pl.BlockSpec((1, emb), lambda i, r: (0, 0)),
        ],
        out_specs=pl.BlockSpec((1, 1, emb), lambda i, r: (i, 0, 0)),
        out_shape=jax.ShapeDtypeStruct((b, 1, emb), jnp.float32),
    )(cat, wf, bf2).reshape(b, emb)


def kernel(data, numpoints, W0, b0, W_blocks, b_blocks, Wf, bf):
    b, c0, n = data.shape
    nb = W_blocks.shape[0] + 1

    xt = jnp.transpose(data, (0, 2, 1))
    x = jnp.pad(xt, ((0, 0), (0, 0), (0, CH - c0)))
    w_first = jnp.concatenate(
        [jnp.pad(W0[:c0], ((0, CH - c0), (0, 0))),
         jnp.pad(W0[c0:], ((0, CH - c0), (0, 0)))], axis=0)
    ws = [w_first] + [W_blocks[i] for i in range(nb - 1)]
    bs = [b0.reshape(1, CH)] + [b_blocks[i].reshape(1, CH) for i in range(nb - 1)]
    sq = jnp.sum(xt * xt, axis=-1)

    feats = []
    for i in range(nb):
        idx = _knn(x, sq.reshape(b, n, 1), sq.reshape(b, 1, n))
        nbr = _gather_rows(x.reshape(b * n, CH),
                           idx.reshape(b * n // PPT, CPT, PPT))
        h2 = _edge_conv(x.reshape(b * n, CH), nbr.reshape(b * n, K, CH),
                        ws[i], bs[i])
        x = h2.reshape(b, n, CH)
        sq = jnp.sum(x * x, axis=-1)
        feats.append(x)

    cat = jnp.concatenate(feats, axis=-1)
    f = _final(cat, Wf, bf.reshape(1, -1))
    return (data, f)


# speedup vs baseline: 11.8803x; 1.1727x over previous
import functools

import jax
import jax.numpy as jnp
from jax import lax
from jax.experimental import pallas as pl
from jax.experimental.pallas import tpu as pltpu
from jax.experimental.pallas import tpu_sc as plsc

K = 16
CH = 64
RB = 256
PPT = 128
CPT = 16
BIG = 3.0e38


def _knn_body(x_blk_ref, x_full_ref, sqc_ref, sqr_ref, idx_ref):
    b = pl.program_id(0)
    n = x_full_ref.shape[1]
    x_blk = x_blk_ref[0]
    x_full = x_full_ref[0]

    g = lax.dot_general(x_blk, x_full, (((1,), (1,)), ((), ())),
                        preferred_element_type=jnp.float32)
    d = (sqc_ref[0] - 2.0 * g) + sqr_ref[0]

    iota = lax.broadcasted_iota(jnp.int32, (RB, n), 1).astype(jnp.float32)
    kiota = lax.broadcasted_iota(jnp.int32, (RB, K), 1)
    idx_blk = jnp.zeros((RB, K), jnp.int32)
    for k in range(K):
        dmin = jnp.min(d, axis=1, keepdims=True)
        cand = jnp.where(d == dmin, iota, BIG)
        imin = jnp.min(cand, axis=1, keepdims=True)
        idx_blk = jnp.where(kiota == k, imin.astype(jnp.int32), idx_blk)
        d = jnp.where(iota == imin, BIG, d)
    idx_ref[0, 0] = idx_blk + b * n


def _knn(x, sq_col, sq_row):
    b, n, _ = x.shape
    nrb = n // RB
    return pl.pallas_call(
        _knn_body,
        grid=(b, nrb),
        in_specs=[
            pl.BlockSpec((1, RB, CH), lambda i, r: (i, r, 0)),
            pl.BlockSpec((1, n, CH), lambda i, r: (i, 0, 0)),
            pl.BlockSpec((1, RB, 1), lambda i, r: (i, r, 0)),
            pl.BlockSpec((1, 1, n), lambda i, r: (i, 0, 0)),
        ],
        out_specs=pl.BlockSpec((1, 1, RB, K), lambda i, r: (i, r, 0, 0)),
        out_shape=jax.ShapeDtypeStruct((b, nrb, RB, K), jnp.int32),
    )(x, x, sq_col, sq_row)


def _sc_body(x_hbm, idx_hbm, out_hbm, idxc_v, rows_v, sem):
    wid = lax.axis_index("s") * 2 + lax.axis_index("c")
    base = wid * PPT

    def chunk(c, carry):
        pltpu.sync_copy(idx_hbm.at[wid, c], idxc_v)
        pltpu.async_copy(x_hbm.at[idxc_v], rows_v, sem).wait()
        pltpu.sync_copy(rows_v, out_hbm.at[pl.ds((base + c * 8) * K, PPT)])
        return carry

    lax.fori_loop(0, CPT, chunk, 0)


@functools.lru_cache(maxsize=None)
def _sc_gather(bn):
    return pl.kernel(
        _sc_body,
        out_type=jax.ShapeDtypeStruct((bn * K, CH), jnp.float32),
        compiler_params=pltpu.CompilerParams(use_tc_tiling_on_sc=False),
        mesh=plsc.VectorSubcoreMesh(core_axis_name="c", subcore_axis_name="s"),
        scratch_types=[
            pltpu.VMEM((PPT,), jnp.int32),
            pltpu.VMEM((PPT, CH), jnp.float32),
            pltpu.SemaphoreType.DMA,
        ],
    )


def _gather_rows(x2, idx3):
    return _sc_gather(x2.shape[0])(x2, idx3)


def _ec_body(x_ref, nbr_ref, w_ref, b_ref, h_ref):
    xc = x_ref[...]
    m = None
    for j in range(K):
        nbj = nbr_ref[:, j, :]
        feat = jnp.concatenate([xc, nbj - xc], axis=1)
        hj = lax.dot_general(feat, w_ref[...], (((1,), (0,)), ((), ())),
                             preferred_element_type=jnp.float32) + b_ref[...]
        m = hj if m is None else jnp.maximum(m, hj)
    h_ref[...] = jnp.maximum(m, 0.0)


def _edge_conv(x2, nbr3, w, b2):
    bn = x2.shape[0]
    return pl.pallas_call(
        _ec_body,
        grid=(bn // RB,),
        in_specs=[
            pl.BlockSpec((RB, CH), lambda r: (r, 0)),
            pl.BlockSpec((RB, K, CH), lambda r: (r, 0, 0)),
            pl.BlockSpec((2 * CH, CH), lambda r: (0, 0)),
            pl.BlockSpec((1, CH), lambda r: (0, 0)),
        ],
        out_specs=pl.BlockSpec((RB, CH), lambda r: (r, 0)),
        out_shape=jax.ShapeDtypeStruct((bn, CH), jnp.float32),
    )(x2, nbr3, w, b2)


def _final_body(cat_ref, wf_ref, bf_ref, out_ref):
    r = pl.program_id(1)
    g = lax.dot_general(cat_ref[0], wf_ref[...], (((1,), (0,)), ((), ())),
                        preferred_element_type=jnp.float32)
    g = jnp.maximum(g + bf_ref[...], 0.0)
    m = jnp.max(g, axis=0, keepdims=True)

    @pl.when(r == 0)
    def _():
        out_ref[0] = m

    @pl.when(r > 0)
    def _():
        out_ref[0] = jnp.maximum(out_ref[0], m)


def _final(cat, wf, bf2):
    b, n, cin = cat.shape
    emb = wf.shape[1]
    return pl.pallas_call(
        _final_body,
        grid=(b, n // RB),
        in_specs=[
            pl.BlockSpec((1, RB, cin), lambda i, r: (i, r, 0)),
            pl.BlockSpec((cin, emb), lambda i, r: (0, 0)),
            pl.BlockSpec((1, emb), lambda i, r: (0, 0)),
        ],
        out_specs=pl.BlockSpec((1, 1, emb), lambda i, r: (i, 0, 0)),
        out_shape=jax.ShapeDtypeStruct((b, 1, emb), jnp.float32),
    )(cat, wf, bf2).reshape(b, emb)


def kernel(data, numpoints, W0, b0, W_blocks, b_blocks, Wf, bf):
    b, c0, n = data.shape
    nb = W_blocks.shape[0] + 1

    xt = jnp.transpose(data, (0, 2, 1))
    x = jnp.pad(xt, ((0, 0), (0, 0), (0, CH - c0)))
    w_first = jnp.concatenate(
        [jnp.pad(W0[:c0], ((0, CH - c0), (0, 0))),
         jnp.pad(W0[c0:], ((0, CH - c0), (0, 0)))], axis=0)
    ws = [w_first] + [W_blocks[i] for i in range(nb - 1)]
    bs = [b0.reshape(1, CH)] + [b_blocks[i].reshape(1, CH) for i in range(nb - 1)]
    sq = jnp.sum(xt * xt, axis=-1)

    feats = []
    for i in range(nb):
        idx = _knn(x, sq.reshape(b, n, 1), sq.reshape(b, 1, n))
        nbr = _gather_rows(x.reshape(b * n, CH),
                           idx.reshape(b * n // PPT, CPT, PPT))
        h2 = _edge_conv(x.reshape(b * n, CH), nbr.reshape(b * n, K, CH),
                        ws[i], bs[i])
        x = h2.reshape(b, n, CH)
        sq = jnp.sum(x * x, axis=-1)
        feats.append(x)

    cat = jnp.concatenate(feats, axis=-1)
    f = _final(cat, Wf, bf.reshape(1, -1))
    return (data, f)


# speedup vs baseline: 12.7109x; 1.0699x over previous
import functools

import jax
import jax.numpy as jnp
from jax import lax
from jax.experimental import pallas as pl
from jax.experimental.pallas import tpu as pltpu
from jax.experimental.pallas import tpu_sc as plsc

K = 16
CH = 64
RB = 256
BIG = 3.0e38


def _knn_body(x_blk_ref, x_full_ref, sqc_ref, sqr_ref, idx_ref):
    b = pl.program_id(0)
    n = x_full_ref.shape[1]
    x_blk = x_blk_ref[0]
    x_full = x_full_ref[0]

    g = lax.dot_general(x_blk, x_full, (((1,), (1,)), ((), ())),
                        preferred_element_type=jnp.float32)
    d = (sqc_ref[0] - 2.0 * g) + sqr_ref[0]

    iota = lax.broadcasted_iota(jnp.int32, (RB, n), 1).astype(jnp.float32)
    kiota = lax.broadcasted_iota(jnp.int32, (RB, K), 1)
    idx_blk = jnp.zeros((RB, K), jnp.int32)
    for k in range(K):
        dmin = jnp.min(d, axis=1, keepdims=True)
        cand = jnp.where(d == dmin, iota, BIG)
        imin = jnp.min(cand, axis=1, keepdims=True)
        idx_blk = jnp.where(kiota == k, imin.astype(jnp.int32), idx_blk)
        d = jnp.where(iota == imin, BIG, d)
    idx_ref[0, 0] = idx_blk + b * n


def _knn(x, sq_col, sq_row):
    b, n, _ = x.shape
    nrb = n // RB
    return pl.pallas_call(
        _knn_body,
        grid=(b, nrb),
        in_specs=[
            pl.BlockSpec((1, RB, CH), lambda i, r: (i, r, 0)),
            pl.BlockSpec((1, n, CH), lambda i, r: (i, 0, 0)),
            pl.BlockSpec((1, RB, 1), lambda i, r: (i, r, 0)),
            pl.BlockSpec((1, 1, n), lambda i, r: (i, 0, 0)),
        ],
        out_specs=pl.BlockSpec((1, 1, RB, K), lambda i, r: (i, r, 0, 0)),
        out_shape=jax.ShapeDtypeStruct((b, nrb, RB, K), jnp.int32),
    )(x, x, sq_col, sq_row)


def _sc_body(cpt, x_hbm, idx_hbm, out_hbm, idxc_v, rows_v, sem):
    ppt = cpt * 8
    wid = lax.axis_index("s") * 2 + lax.axis_index("c")
    base = wid * ppt

    def chunk(c, carry):
        pltpu.sync_copy(idx_hbm.at[wid, c], idxc_v)
        pltpu.async_copy(x_hbm.at[idxc_v], rows_v, sem).wait()
        pltpu.sync_copy(rows_v, out_hbm.at[pl.ds((base + c * 8) * K, 128)])
        return carry

    lax.fori_loop(0, cpt, chunk, 0)


@functools.lru_cache(maxsize=None)
def _sc_gather(bn):
    cpt = bn * K // (32 * 128)
    return pl.kernel(
        functools.partial(_sc_body, cpt),
        out_type=jax.ShapeDtypeStruct((bn * K, CH), jnp.float32),
        compiler_params=pltpu.CompilerParams(use_tc_tiling_on_sc=False),
        mesh=plsc.VectorSubcoreMesh(core_axis_name="c", subcore_axis_name="s"),
        scratch_types=[
            pltpu.VMEM((128,), jnp.int32),
            pltpu.VMEM((128, CH), jnp.float32),
            pltpu.SemaphoreType.DMA,
        ],
    )


def _gather_rows(x2, idx):
    bn = x2.shape[0]
    cpt = bn * K // (32 * 128)
    return _sc_gather(bn)(x2, idx.reshape(32, cpt, 128))


def _ec_body(x_ref, nbr_ref, w_ref, b_ref, h_ref):
    xc = x_ref[...]
    m = None
    for j in range(K):
        nbj = nbr_ref[:, j, :]
        feat = jnp.concatenate([xc, nbj - xc], axis=1)
        hj = lax.dot_general(feat, w_ref[...], (((1,), (0,)), ((), ())),
                             preferred_element_type=jnp.float32) + b_ref[...]
        m = hj if m is None else jnp.maximum(m, hj)
    h_ref[...] = jnp.maximum(m, 0.0)


def _edge_conv(x2, nbr3, w, b2):
    bn = x2.shape[0]
    return pl.pallas_call(
        _ec_body,
        grid=(bn // RB,),
        in_specs=[
            pl.BlockSpec((RB, CH), lambda r: (r, 0)),
            pl.BlockSpec((RB, K, CH), lambda r: (r, 0, 0)),
            pl.BlockSpec((2 * CH, CH), lambda r: (0, 0)),
            pl.BlockSpec((1, CH), lambda r: (0, 0)),
        ],
        out_specs=pl.BlockSpec((RB, CH), lambda r: (r, 0)),
        out_shape=jax.ShapeDtypeStruct((bn, CH), jnp.float32),
    )(x2, nbr3, w, b2)


def _final_body(cat_ref, wf_ref, bf_ref, out_ref):
    r = pl.program_id(1)
    g = lax.dot_general(cat_ref[0], wf_ref[...], (((1,), (0,)), ((), ())),
                        preferred_element_type=jnp.float32)
    g = jnp.maximum(g + bf_ref[...], 0.0)
    m = jnp.max(g, axis=0, keepdims=True)

    @pl.when(r == 0)
    def _():
        out_ref[0] = m

    @pl.when(r > 0)
    def _():
        out_ref[0] = jnp.maximum(out_ref[0], m)


def _final(cat, wf, bf2):
    b, n, cin = cat.shape
    emb = wf.shape[1]
    return pl.pallas_call(
        _final_body,
        grid=(b, n // RB),
        in_specs=[
            pl.BlockSpec((1, RB, cin), lambda i, r: (i, r, 0)),
            pl.BlockSpec((cin, emb), lambda i, r: (0, 0)),
            pl.BlockSpec((1, emb), lambda i, r: (0, 0)),
        ],
        out_specs=pl.BlockSpec((1, 1, emb), lambda i, r: (i, 0, 0)),
        out_shape=jax.ShapeDtypeStruct((b, 1, emb), jnp.float32),
    )(cat, wf, bf2).reshape(b, emb)


def kernel(data, numpoints, W0, b0, W_blocks, b_blocks, Wf, bf):
    b, c0, n = data.shape
    nb = W_blocks.shape[0] + 1

    xt = jnp.transpose(data, (0, 2, 1))
    x = jnp.pad(xt, ((0, 0), (0, 0), (0, CH - c0)))
    w_first = jnp.concatenate(
        [jnp.pad(W0[:c0], ((0, CH - c0), (0, 0))),
         jnp.pad(W0[c0:], ((0, CH - c0), (0, 0)))], axis=0)
    ws = [w_first] + [W_blocks[i] for i in range(nb - 1)]
    bs = [b0.reshape(1, CH)] + [b_blocks[i].reshape(1, CH) for i in range(nb - 1)]
    sq = jnp.sum(xt * xt, axis=-1)

    xs = [x[bi:bi + 1] for bi in range(b)]
    sqs = [sq[bi:bi + 1] for bi in range(b)]
    feats = [[] for _ in range(b)]
    for i in range(nb):
        for bi in range(b):
            xi, sqi = xs[bi], sqs[bi]
            idx = _knn(xi, sqi.reshape(1, n, 1), sqi.reshape(1, 1, n))
            nbr = _gather_rows(xi.reshape(n, CH), idx)
            h2 = _edge_conv(xi.reshape(n, CH), nbr.reshape(n, K, CH),
                            ws[i], bs[i])
            xs[bi] = h2.reshape(1, n, CH)
            sqs[bi] = jnp.sum(xs[bi] * xs[bi], axis=-1)
            feats[bi].append(xs[bi])

    cat = jnp.concatenate(
        [jnp.concatenate(f, axis=-1) for f in feats], axis=0)
    f = _final(cat, Wf, bf.reshape(1, -1))
    return (data, f)
